# baseline (speedup 1.0000x reference)
import functools

import jax
import jax.numpy as jnp
from jax import lax
from jax.experimental import pallas as pl
from jax.experimental.pallas import tpu as pltpu
from jax.experimental.pallas import tpu_sc as plsc

N_NODES = 10000
N_PAD = 10112
D = 128
NC = 2
NS = 16
NW = NC * NS
CHUNK = 128
N_PASS = 2
ROWS_PER_TILE = N_PAD // NS


def _sc_message_passing(nf, src, dst, w, zeros):
    n_chunks = src.shape[2]
    mesh = plsc.VectorSubcoreMesh(core_axis_name="c", subcore_axis_name="s")

    @functools.partial(
        pl.kernel,
        mesh=mesh,
        out_type=jax.ShapeDtypeStruct((NC, N_PAD, D), jnp.float32),
        scratch_types=[
            pltpu.VMEM((n_chunks, CHUNK), jnp.int32),
            pltpu.VMEM((n_chunks, CHUNK), jnp.int32),
            pltpu.VMEM((n_chunks, CHUNK), jnp.float32),
            pltpu.VMEM((2, CHUNK, D), jnp.float32),
            pltpu.VMEM_SHARED((N_PAD, D), jnp.float32),
            pltpu.SemaphoreType.DMA,
        ],
    )
    def k(nf_hbm, src_hbm, dst_hbm, w_hbm, z_hbm, out_hbm,
          src_v, dst_v, w_v, rows_v, h_sh, sem_g):
        c = lax.axis_index("c")
        s = lax.axis_index("s")
        wid = c * NS + s
        rows_slice = pl.ds(s * ROWS_PER_TILE, ROWS_PER_TILE)

        pltpu.sync_copy(z_hbm, h_sh.at[rows_slice])
        plsc.subcore_barrier()

        def scale(b, j):
            def group_body(g, carry2):
                wg = w_v[j, pl.ds(g * 16, 16)]
                for r16 in range(16):
                    wv = jnp.full((16,), wg[r16], dtype=jnp.float32)
                    r = g * 16 + r16
                    for u in range(D // 16):
                        sl = pl.ds(u * 16, 16)
                        rows_v[b, r, sl] = rows_v[b, r, sl] * wv
                return carry2

            lax.fori_loop(0, CHUNK // 16, group_body, 0)

        def chunk_body(j, carry):
            b = lax.rem(j, 2)
            pltpu.make_async_copy(
                nf_hbm.at[src_v.at[j]], rows_v.at[b], sem_g).wait()
            scale(b, j)
            pltpu.sync_copy(rows_v.at[b], h_sh.at[dst_v.at[j]], add=True)

            @pl.when(j + 2 < n_chunks)
            def _():
                pltpu.async_copy(
                    nf_hbm.at[src_v.at[j + 2]], rows_v.at[b], sem_g)

            return carry

        for p in range(N_PASS):
            pltpu.sync_copy(src_hbm.at[wid, p], src_v)
            pltpu.sync_copy(dst_hbm.at[wid, p], dst_v)
            pltpu.sync_copy(w_hbm.at[wid, p], w_v)
            pltpu.async_copy(nf_hbm.at[src_v.at[0]], rows_v.at[0], sem_g)
            pltpu.async_copy(nf_hbm.at[src_v.at[1]], rows_v.at[1], sem_g)
            lax.fori_loop(0, n_chunks, chunk_body, 0)

        plsc.subcore_barrier()
        pltpu.sync_copy(h_sh.at[rows_slice], out_hbm.at[c, rows_slice])

    return k(nf, src, dst, w, zeros)


def _tc_linear(hparts, W, b):
    blk = 1000
    grid = N_NODES // blk

    def body(h_ref, w_ref, b_ref, o_ref):
        h = h_ref[0] + h_ref[1]
        o_ref[...] = lax.dot_general(
            h, w_ref[...], (((1,), (1,)), ((), ())),
            preferred_element_type=jnp.float32) + b_ref[...]

    return pl.pallas_call(
        body,
        grid=(grid,),
        in_specs=[
            pl.BlockSpec((NC, blk, D), lambda i: (0, i, 0)),
            pl.BlockSpec((D, D), lambda i: (0, 0)),
            pl.BlockSpec((1, D), lambda i: (0, 0)),
        ],
        out_specs=pl.BlockSpec((blk, D), lambda i: (i, 0)),
        out_shape=jax.ShapeDtypeStruct((N_NODES, D), jnp.float32),
    )(hparts, W, b.reshape(1, D))


def kernel(node_features, edge_index, edge_weights, W, b):
    e = edge_index.shape[1]
    src = edge_index[0].astype(jnp.int32)
    dst = edge_index[1].astype(jnp.int32)
    w = edge_weights.astype(jnp.float32)
    quantum = NW * N_PASS * CHUNK
    per_w = -(-e // quantum) * N_PASS * CHUNK
    pad = NW * per_w - e
    src = jnp.concatenate([src, jnp.zeros((pad,), jnp.int32)])
    dst = jnp.concatenate([dst, jnp.zeros((pad,), jnp.int32)])
    w = jnp.concatenate([w, jnp.zeros((pad,), jnp.float32)])
    shape = (NW, N_PASS, per_w // (N_PASS * CHUNK), CHUNK)
    src, dst, w = (x.reshape(shape) for x in (src, dst, w))
    zeros = jnp.zeros((ROWS_PER_TILE, D), jnp.float32)
    hparts = _sc_message_passing(node_features, src, dst, w, zeros)
    return _tc_linear(hparts, W, b)


# speedup vs baseline: 1.3354x; 1.3354x over previous
import functools

import jax
import jax.numpy as jnp
from jax import lax
from jax.experimental import pallas as pl
from jax.experimental.pallas import tpu as pltpu
from jax.experimental.pallas import tpu_sc as plsc

N_NODES = 10000
N_PAD = 10240
D = 128
NC = 2
NS = 16
NW = NC * NS
CHUNK = 128
ROWS_PER_TILE = N_PAD // NS


def _sc_message_passing(nf, src, dst, w, zeros):
    n_chunks = src.shape[1]
    mesh = plsc.VectorSubcoreMesh(core_axis_name="c", subcore_axis_name="s")

    @functools.partial(
        pl.kernel,
        mesh=mesh,
        out_type=jax.ShapeDtypeStruct((NC, N_PAD, D), jnp.float32),
        scratch_types=[
            pltpu.VMEM((n_chunks, CHUNK), jnp.int32),
            pltpu.VMEM((n_chunks, CHUNK), jnp.int32),
            pltpu.VMEM((n_chunks, CHUNK), jnp.float32),
            pltpu.VMEM((CHUNK, D), jnp.float32),
            pltpu.VMEM_SHARED((N_PAD, D), jnp.float32),
            pltpu.SemaphoreType.DMA,
        ],
    )
    def k(nf_hbm, src_hbm, dst_hbm, w_hbm, z_hbm, out_hbm,
          src_v, dst_v, w_v, rows_v, h_sh, sem_g):
        c = lax.axis_index("c")
        s = lax.axis_index("s")
        wid = c * NS + s

        pltpu.sync_copy(z_hbm, h_sh.at[pl.ds(s * ROWS_PER_TILE, ROWS_PER_TILE)])
        pltpu.sync_copy(src_hbm.at[wid], src_v)
        pltpu.sync_copy(dst_hbm.at[wid], dst_v)
        pltpu.sync_copy(w_hbm.at[wid], w_v)
        plsc.subcore_barrier()

        def chunk_body(j, carry):
            pltpu.async_copy(nf_hbm.at[src_v.at[j]], rows_v, sem_g).wait()

            def group_body(g, carry2):
                wg = w_v[j, pl.ds(g * 16, 16)]
                for r16 in range(16):
                    wv = jnp.full((16,), wg[r16], dtype=jnp.float32)
                    r = g * 16 + r16
                    for u in range(D // 16):
                        sl = pl.ds(u * 16, 16)
                        rows_v[r, sl] = rows_v[r, sl] * wv
                return carry2

            lax.fori_loop(0, CHUNK // 16, group_body, 0)
            pltpu.sync_copy(rows_v, h_sh.at[dst_v.at[j]], add=True)
            return carry

        lax.fori_loop(0, n_chunks, chunk_body, 0)
        plsc.subcore_barrier()
        pltpu.sync_copy(h_sh.at[pl.ds(s * ROWS_PER_TILE, ROWS_PER_TILE)],
                        out_hbm.at[c, pl.ds(s * ROWS_PER_TILE, ROWS_PER_TILE)])

    return k(nf, src, dst, w, zeros)


def _tc_linear(hparts, W, b):
    blk = 1000
    grid = N_NODES // blk

    def body(h_ref, w_ref, b_ref, o_ref):
        h = h_ref[0] + h_ref[1]
        o_ref[...] = lax.dot_general(
            h, w_ref[...], (((1,), (1,)), ((), ())),
            preferred_element_type=jnp.float32) + b_ref[...]

    return pl.pallas_call(
        body,
        grid=(grid,),
        in_specs=[
            pl.BlockSpec((NC, blk, D), lambda i: (0, i, 0)),
            pl.BlockSpec((D, D), lambda i: (0, 0)),
            pl.BlockSpec((1, D), lambda i: (0, 0)),
        ],
        out_specs=pl.BlockSpec((blk, D), lambda i: (i, 0)),
        out_shape=jax.ShapeDtypeStruct((N_NODES, D), jnp.float32),
    )(hparts, W, b.reshape(1, D))


def kernel(node_features, edge_index, edge_weights, W, b):
    e = edge_index.shape[1]
    src = edge_index[0].astype(jnp.int32)
    dst = edge_index[1].astype(jnp.int32)
    w = edge_weights.astype(jnp.float32)
    per_w = -(-e // (NW * CHUNK)) * CHUNK
    pad = NW * per_w - e
    src = jnp.concatenate([src, jnp.zeros((pad,), jnp.int32)])
    dst = jnp.concatenate([dst, jnp.zeros((pad,), jnp.int32)])
    w = jnp.concatenate([w, jnp.zeros((pad,), jnp.float32)])
    src = src.reshape(NW, per_w // CHUNK, CHUNK)
    dst = dst.reshape(NW, per_w // CHUNK, CHUNK)
    w = w.reshape(NW, per_w // CHUNK, CHUNK)
    zeros = jnp.zeros((ROWS_PER_TILE, D), jnp.float32)
    hparts = _sc_message_passing(node_features, src, dst, w, zeros)
    return _tc_linear(hparts, W, b)


# speedup vs baseline: 1.5186x; 1.1372x over previous
import functools

import jax
import jax.numpy as jnp
from jax import lax
from jax.experimental import pallas as pl
from jax.experimental.pallas import tpu as pltpu
from jax.experimental.pallas import tpu_sc as plsc

N_NODES = 10000
N_PAD = 10240
D = 128
NC = 2
NS = 16
NW = NC * NS
CHUNK = 128
ROWS_PER_TILE = N_PAD // NS


def _sc_message_passing(nf, src, dst, w, zeros):
    n_chunks = src.shape[1]
    mesh = plsc.VectorSubcoreMesh(core_axis_name="c", subcore_axis_name="s")

    @functools.partial(
        pl.kernel,
        mesh=mesh,
        out_type=jax.ShapeDtypeStruct((NC, N_PAD, D), jnp.float32),
        scratch_types=[
            pltpu.VMEM((n_chunks, CHUNK), jnp.int32),
            pltpu.VMEM((n_chunks, CHUNK), jnp.int32),
            pltpu.VMEM((n_chunks, CHUNK), jnp.float32),
            pltpu.VMEM((CHUNK, D), jnp.float32),
            pltpu.VMEM_SHARED((N_PAD, D), jnp.float32),
            pltpu.SemaphoreType.DMA,
        ],
    )
    def k(nf_hbm, src_hbm, dst_hbm, w_hbm, z_hbm, out_hbm,
          src_v, dst_v, w_v, rows_v, h_sh, sem_g):
        c = lax.axis_index("c")
        s = lax.axis_index("s")
        wid = c * NS + s

        pltpu.sync_copy(z_hbm, h_sh.at[pl.ds(s * ROWS_PER_TILE, ROWS_PER_TILE)])
        pltpu.sync_copy(src_hbm.at[wid], src_v)
        pltpu.sync_copy(dst_hbm.at[wid], dst_v)
        pltpu.sync_copy(w_hbm.at[wid], w_v)
        plsc.subcore_barrier()

        def chunk_body(j, carry):
            pltpu.async_copy(nf_hbm.at[src_v.at[j]], rows_v, sem_g).wait()

            @functools.partial(
                plsc.parallel_loop, 0, CHUNK // 16, unroll=2)
            def group_body(g):
                wg = w_v[j, pl.ds(g * 16, 16)]
                for r16 in range(16):
                    wv = jnp.full((16,), wg[r16], dtype=jnp.float32)
                    r = g * 16 + r16
                    for u in range(D // 16):
                        sl = pl.ds(u * 16, 16)
                        rows_v[r, sl] = rows_v[r, sl] * wv
            pltpu.sync_copy(rows_v, h_sh.at[dst_v.at[j]], add=True)
            return carry

        lax.fori_loop(0, n_chunks, chunk_body, 0)
        plsc.subcore_barrier()
        pltpu.sync_copy(h_sh.at[pl.ds(s * ROWS_PER_TILE, ROWS_PER_TILE)],
                        out_hbm.at[c, pl.ds(s * ROWS_PER_TILE, ROWS_PER_TILE)])

    return k(nf, src, dst, w, zeros)


def _tc_linear(hparts, W, b):
    blk = 1000
    grid = N_NODES // blk

    def body(h_ref, w_ref, b_ref, o_ref):
        h = h_ref[0] + h_ref[1]
        o_ref[...] = lax.dot_general(
            h, w_ref[...], (((1,), (1,)), ((), ())),
            preferred_element_type=jnp.float32) + b_ref[...]

    return pl.pallas_call(
        body,
        grid=(grid,),
        in_specs=[
            pl.BlockSpec((NC, blk, D), lambda i: (0, i, 0)),
            pl.BlockSpec((D, D), lambda i: (0, 0)),
            pl.BlockSpec((1, D), lambda i: (0, 0)),
        ],
        out_specs=pl.BlockSpec((blk, D), lambda i: (i, 0)),
        out_shape=jax.ShapeDtypeStruct((N_NODES, D), jnp.float32),
    )(hparts, W, b.reshape(1, D))


def kernel(node_features, edge_index, edge_weights, W, b):
    e = edge_index.shape[1]
    src = edge_index[0].astype(jnp.int32)
    dst = edge_index[1].astype(jnp.int32)
    w = edge_weights.astype(jnp.float32)
    per_w = -(-e // (NW * CHUNK)) * CHUNK
    pad = NW * per_w - e
    src = jnp.concatenate([src, jnp.zeros((pad,), jnp.int32)])
    dst = jnp.concatenate([dst, jnp.zeros((pad,), jnp.int32)])
    w = jnp.concatenate([w, jnp.zeros((pad,), jnp.float32)])
    src = src.reshape(NW, per_w // CHUNK, CHUNK)
    dst = dst.reshape(NW, per_w // CHUNK, CHUNK)
    w = w.reshape(NW, per_w // CHUNK, CHUNK)
    zeros = jnp.zeros((ROWS_PER_TILE, D), jnp.float32)
    hparts = _sc_message_passing(node_features, src, dst, w, zeros)
    return _tc_linear(hparts, W, b)
